# Initial kernel scaffold; baseline (speedup 1.0000x reference)
import jax
import jax.numpy as jnp
from jax.experimental import pallas as pl


def kernel(x, edge_index, W_l0, b_l0, W_r0, gamma0, beta0, W_l1, b_l1, W_r1, gamma1, beta1, W_l2, b_l2, W_r2, gamma2, beta2):
    raise NotImplementedError("write your pallas kernel here")


# speedup vs baseline: 3.9345x; 3.9345x over previous
import functools

import jax
import jax.numpy as jnp
from jax import lax
from jax.experimental import pallas as pl
from jax.experimental.pallas import tpu as pltpu
from jax.experimental.pallas import tpu_sc as plsc

NC, NS, LANES = 2, 16, 16
NW = NC * NS
D = 128
CH = 128
CNTW = 128

_mesh = plsc.VectorSubcoreMesh(
    core_axis_name="c", subcore_axis_name="s", num_cores=NC, num_subcores=NS)


def _npad(n):
  per_tile = pl.cdiv(n + 1, NS)
  per_tile = (per_tile + 7) // 8 * 8
  return per_tile * NS


def _make_sums_kernel(e_pad, npad):
  nchunk = e_pad // (NW * CH)
  rows_per_tile = npad // NS

  @functools.partial(
      pl.kernel,
      out_type=jax.ShapeDtypeStruct((NC * npad, D), jnp.float32),
      mesh=_mesh,
      scratch_types=[
          pltpu.VMEM((CH,), jnp.int32),
          pltpu.VMEM((CH,), jnp.int32),
          pltpu.VMEM((CH, D), jnp.float32),
          pltpu.VMEM_SHARED((npad, D), jnp.float32),
          pltpu.SemaphoreType.DMA,
      ],
  )
  def sums(src_hbm, dst_hbm, h_hbm, z_hbm, out_hbm, sidx, didx, rows, accum,
           sem):
    cid = lax.axis_index("c")
    sid = lax.axis_index("s")
    rslice = pl.ds(sid * rows_per_tile, rows_per_tile)
    pltpu.sync_copy(z_hbm.at[rslice], accum.at[rslice])
    plsc.subcore_barrier()

    tile = cid * NS + sid

    def body(j, carry):
      base = pl.multiple_of((tile * nchunk + j) * CH, CH)
      pltpu.sync_copy(src_hbm.at[pl.ds(base, CH)], sidx)
      pltpu.sync_copy(dst_hbm.at[pl.ds(base, CH)], didx)
      pltpu.async_copy(h_hbm.at[sidx], rows, sem).wait()
      pltpu.sync_copy(rows, accum.at[didx], add=True)
      return carry

    lax.fori_loop(0, nchunk, body, 0)
    plsc.subcore_barrier()
    out_base = pl.multiple_of(cid * npad + sid * rows_per_tile, 8)
    pltpu.sync_copy(accum.at[rslice],
                    out_hbm.at[pl.ds(out_base, rows_per_tile)])

  return sums


def _make_counts_kernel(e_pad, npad):
  nchunk = e_pad // (NW * CH)
  rows_per_tile = npad // NS

  @functools.partial(
      pl.kernel,
      out_type=jax.ShapeDtypeStruct((NC * npad, CNTW), jnp.float32),
      mesh=_mesh,
      scratch_types=[
          pltpu.VMEM((CH,), jnp.int32),
          pltpu.VMEM((CH, CNTW), jnp.float32),
          pltpu.VMEM_SHARED((npad, CNTW), jnp.float32),
      ],
  )
  def counts(dst_hbm, ones_hbm, z_hbm, out_hbm, didx, ones_v, accum):
    cid = lax.axis_index("c")
    sid = lax.axis_index("s")
    rslice = pl.ds(sid * rows_per_tile, rows_per_tile)
    pltpu.sync_copy(z_hbm.at[rslice], accum.at[rslice])
    pltpu.sync_copy(ones_hbm, ones_v)
    plsc.subcore_barrier()

    tile = cid * NS + sid

    def body(j, carry):
      base = pl.multiple_of((tile * nchunk + j) * CH, CH)
      pltpu.sync_copy(dst_hbm.at[pl.ds(base, CH)], didx)
      pltpu.sync_copy(ones_v, accum.at[didx], add=True)
      return carry

    lax.fori_loop(0, nchunk, body, 0)
    plsc.subcore_barrier()
    out_base = pl.multiple_of(cid * npad + sid * rows_per_tile, 8)
    pltpu.sync_copy(accum.at[rslice],
                    out_hbm.at[pl.ds(out_base, rows_per_tile)])

  return counts


def _dense_body(p_ref, cnt_ref, h_ref, wl_ref, bl_ref, wr_ref, g_ref, be_ref,
                out_ref, *, n, npad):
  p = p_ref[:n, :] + p_ref[npad:npad + n, :]
  cnt = cnt_ref[:n, 0:1] + cnt_ref[npad:npad + n, 0:1]
  agg = p / jnp.maximum(cnt, 1.0)
  z = (jnp.dot(agg, wl_ref[...], preferred_element_type=jnp.float32)
       + bl_ref[...][None, :]
       + jnp.dot(h_ref[...], wr_ref[...], preferred_element_type=jnp.float32))
  mu = jnp.mean(z, axis=0, keepdims=True)
  zc = z - mu
  var = jnp.mean(zc * zc, axis=0, keepdims=True)
  y = g_ref[...][None, :] * zc * lax.rsqrt(var + 1e-5) + be_ref[...][None, :]
  out_ref[...] = jnp.where(y > 0.0, y, jnp.exp(jnp.minimum(y, 0.0)) - 1.0)


def _make_dense_kernel(n, npad):
  return pl.pallas_call(
      functools.partial(_dense_body, n=n, npad=npad),
      out_shape=jax.ShapeDtypeStruct((n, D), jnp.float32),
  )


def kernel(x, edge_index, W_l0, b_l0, W_r0, gamma0, beta0, W_l1, b_l1, W_r1,
           gamma1, beta1, W_l2, b_l2, W_r2, gamma2, beta2):
  n = x.shape[0]
  e = edge_index.shape[1]
  npad = _npad(n)
  e_pad = pl.cdiv(e, NW * CH) * NW * CH

  src = edge_index[0].astype(jnp.int32)
  dst = edge_index[1].astype(jnp.int32)
  pad = e_pad - e
  if pad:
    src = jnp.concatenate([src, jnp.zeros((pad,), jnp.int32)])
    dst = jnp.concatenate([dst, jnp.full((pad,), n, jnp.int32)])

  zsum = jnp.zeros((npad, D), jnp.float32)
  zcnt = jnp.zeros((npad, CNTW), jnp.float32)
  ones_rows = jnp.ones((CH, CNTW), jnp.float32)

  sums_k = _make_sums_kernel(e_pad, npad)
  counts_k = _make_counts_kernel(e_pad, npad)
  dense_k = _make_dense_kernel(n, npad)

  cnt_parts = counts_k(dst, ones_rows, zcnt)

  h = x
  for (wl, bl, wr, g, b) in (
      (W_l0, b_l0, W_r0, gamma0, beta0),
      (W_l1, b_l1, W_r1, gamma1, beta1),
      (W_l2, b_l2, W_r2, gamma2, beta2),
  ):
    parts = sums_k(src, dst, h, zsum)
    h = dense_k(parts, cnt_parts, h, wl, bl, wr, g, b)
  return h
